# baseline (speedup 1.0000x reference)
import functools

import jax
import jax.numpy as jnp
from jax.experimental import pallas as pl
from jax.experimental.pallas import tpu as pltpu

_N = 8192
_D = 1024
_E = 16
_BT = 4096
_NB = _N // _BT


def _float_key(x):
    u = jax.lax.bitcast_convert_type(x, jnp.int32)
    return u ^ ((u >> 31) & jnp.int32(0x7FFFFFFF))


def _gate_kernel(h_ref, w_ref, fw_ref, lsim_ref, logits_ref, logits_t_ref,
                 pp_ref, *, K):
    i = pl.program_id(0)
    h = h_ref[...]
    w = w_ref[...]
    lb = jax.lax.dot_general(
        h, w, (((1,), (1,)), ((), ())), preferred_element_type=jnp.float32
    )
    logits_ref[pl.ds(i * _BT, _BT), :] = lb
    lb_t = jnp.transpose(lb)
    logits_t_ref[:, pl.ds(i * _BT, _BT)] = lb_t

    mx = jnp.max(lb_t, axis=0, keepdims=True)
    ex = jnp.exp(lb_t - mx)
    p_t = ex / jnp.sum(ex, axis=0, keepdims=True)
    pp_blk = jax.lax.dot_general(
        p_t, p_t, (((1,), (1,)), ((), ())), preferred_element_type=jnp.float32
    )

    @pl.when(i == 0)
    def _init():
        pp_ref[...] = pp_blk

    @pl.when(i > 0)
    def _acc():
        pp_ref[...] += pp_blk

    @pl.when(i == _NB - 1)
    def _finalize():
        logits_t = logits_t_ref[...]
        key = _float_key(logits_t)

        def count_ge(t):
            return jnp.sum(jnp.where(key >= t, jnp.int32(1), jnp.int32(0)))

        def _i32(x):
            x &= 0xFFFFFFFF
            return x - (1 << 32) if x >= (1 << 31) else x

        t = jnp.int32(-2147483648)
        cands = [jnp.int32(_i32((j << 28) ^ 0x80000000)) for j in range(1, 16)]
        counts = [count_ge(c) for c in cands]
        for c, n in zip(cands, counts):
            t = jnp.where(n >= K, c, t)
        for p in (24, 20, 16, 12, 8, 4, 0):
            cands = [t | jnp.int32(j << p) for j in range(1, 16)]
            counts = [count_ge(c) for c in cands]
            for c, n in zip(cands, counts):
                t = jnp.where(n >= K, c, t)
        kth_key = t
        kth_val = jax.lax.bitcast_convert_type(
            kth_key ^ ((kth_key >> 31) & jnp.int32(0x7FFFFFFF)), jnp.float32
        )

        mf_t = (logits_t >= kth_val).astype(jnp.float32)
        mp = jax.lax.dot_general(
            mf_t, mf_t, (((1,), (1,)), ((), ())),
            preferred_element_type=jnp.float32,
        )
        pp = pp_ref[...]

        rows = jax.lax.broadcasted_iota(jnp.int32, (_E, _E), 0)
        cols = jax.lax.broadcasted_iota(jnp.int32, (_E, _E), 1)
        eye = (rows == cols).astype(jnp.float32)

        eps = jnp.float32(jnp.finfo(jnp.float32).eps)
        sum_diag = jnp.sum(mp * eye) + eps
        sum_all = jnp.sum(mp) + eps
        off_factor = jnp.float32(_E * _E - _E) / sum_all
        diag_factor = jnp.float32(_E) / sum_diag
        mpp = mp * pp
        lsim = (jnp.sum(mpp * (1.0 - eye)) * off_factor
                + jnp.sum(mpp * eye) * diag_factor) / jnp.float32(_E)
        lsim_ref[...] = jnp.reshape(lsim, (1, 1))

        logits = logits_ref[...]
        fw_ref[...] = jnp.where(logits >= kth_val, logits, 0.0)


@jax.jit
def kernel(hidden_states, W):
    num_tokens, _ = hidden_states.shape
    K = int(num_tokens * 2.0)
    K = max(1, min(K, num_tokens * W.shape[0]))

    fw, lsim = pl.pallas_call(
        functools.partial(_gate_kernel, K=K),
        grid=(_NB,),
        in_specs=[
            pl.BlockSpec((_BT, _D), lambda i: (i, 0)),
            pl.BlockSpec((_E, _D), lambda i: (0, 0)),
        ],
        out_specs=[
            pl.BlockSpec((_N, _E), lambda i: (0, 0)),
            pl.BlockSpec((1, 1), lambda i: (0, 0)),
        ],
        out_shape=[
            jax.ShapeDtypeStruct((_N, _E), jnp.float32),
            jax.ShapeDtypeStruct((1, 1), jnp.float32),
        ],
        scratch_shapes=[
            pltpu.VMEM((_N, _E), jnp.float32),
            pltpu.VMEM((_E, _N), jnp.float32),
            pltpu.VMEM((_E, _E), jnp.float32),
        ],
    )(hidden_states, W)
    return fw, jnp.reshape(lsim, ())


# speedup vs baseline: 1.0551x; 1.0551x over previous
import functools

import jax
import jax.numpy as jnp
from jax.experimental import pallas as pl
from jax.experimental.pallas import tpu as pltpu

_N = 8192
_D = 1024
_E = 16
_BT = 2048
_NB = _N // _BT


def _float_key(x):
    u = jax.lax.bitcast_convert_type(x, jnp.int32)
    return u ^ ((u >> 31) & jnp.int32(0x7FFFFFFF))


def _i32(x):
    x &= 0xFFFFFFFF
    return x - (1 << 32) if x >= (1 << 31) else x


_PASS0_CANDS = [_i32((j << 28) ^ 0x80000000) for j in range(1, 16)]


def _gate_kernel(h_ref, w_ref, fw_ref, lsim_ref, cnt0_ref, logits_ref,
                 logits_t_ref, pp_ref, *, K):
    i = pl.program_id(0)
    h = h_ref[...]
    w = w_ref[...]
    lb = jax.lax.dot_general(
        h, w, (((1,), (1,)), ((), ())), preferred_element_type=jnp.float32
    )
    logits_ref[pl.ds(i * _BT, _BT), :] = lb
    lb_t = jnp.transpose(lb)
    logits_t_ref[:, pl.ds(i * _BT, _BT)] = lb_t

    mx = jnp.max(lb_t, axis=0, keepdims=True)
    ex = jnp.exp(lb_t - mx)
    p_t = ex / jnp.sum(ex, axis=0, keepdims=True)
    pp_blk = jax.lax.dot_general(
        p_t, p_t, (((1,), (1,)), ((), ())), preferred_element_type=jnp.float32
    )

    key_blk = _float_key(lb_t)
    cnt_blk = [
        jnp.sum(jnp.where(key_blk >= jnp.int32(c), jnp.int32(1), jnp.int32(0)))
        for c in _PASS0_CANDS
    ]

    @pl.when(i == 0)
    def _init():
        pp_ref[...] = pp_blk
        for j, nb in enumerate(cnt_blk):
            cnt0_ref[j] = nb

    @pl.when(i > 0)
    def _acc():
        pp_ref[...] += pp_blk
        for j, nb in enumerate(cnt_blk):
            cnt0_ref[j] = cnt0_ref[j] + nb

    @pl.when(i == _NB - 1)
    def _finalize():
        logits_t = logits_t_ref[...]
        key = _float_key(logits_t)

        def count_ge(t):
            return jnp.sum(jnp.where(key >= t, jnp.int32(1), jnp.int32(0)))

        t = jnp.int32(-2147483648)
        for j, c in enumerate(_PASS0_CANDS):
            t = jnp.where(cnt0_ref[j] >= K, jnp.int32(c), t)
        for p in (24, 20, 16, 12, 8, 4, 0):
            cands = [t | jnp.int32(j << p) for j in range(1, 16)]
            counts = [count_ge(c) for c in cands]
            for c, n in zip(cands, counts):
                t = jnp.where(n >= K, c, t)
        kth_key = t
        kth_val = jax.lax.bitcast_convert_type(
            kth_key ^ ((kth_key >> 31) & jnp.int32(0x7FFFFFFF)), jnp.float32
        )

        mf_t = (logits_t >= kth_val).astype(jnp.float32)
        mp = jax.lax.dot_general(
            mf_t, mf_t, (((1,), (1,)), ((), ())),
            preferred_element_type=jnp.float32,
        )
        pp = pp_ref[...]

        rows = jax.lax.broadcasted_iota(jnp.int32, (_E, _E), 0)
        cols = jax.lax.broadcasted_iota(jnp.int32, (_E, _E), 1)
        eye = (rows == cols).astype(jnp.float32)

        eps = jnp.float32(jnp.finfo(jnp.float32).eps)
        sum_diag = jnp.sum(mp * eye) + eps
        sum_all = jnp.sum(mp) + eps
        off_factor = jnp.float32(_E * _E - _E) / sum_all
        diag_factor = jnp.float32(_E) / sum_diag
        mpp = mp * pp
        lsim = (jnp.sum(mpp * (1.0 - eye)) * off_factor
                + jnp.sum(mpp * eye) * diag_factor) / jnp.float32(_E)
        lsim_ref[...] = jnp.reshape(lsim, (1, 1))

        logits = logits_ref[...]
        fw_ref[...] = jnp.where(logits >= kth_val, logits, 0.0)


@jax.jit
def kernel(hidden_states, W):
    num_tokens, _ = hidden_states.shape
    K = int(num_tokens * 2.0)
    K = max(1, min(K, num_tokens * W.shape[0]))

    fw, lsim = pl.pallas_call(
        functools.partial(_gate_kernel, K=K),
        grid=(_NB,),
        in_specs=[
            pl.BlockSpec((_BT, _D), lambda i: (i, 0)),
            pl.BlockSpec((_E, _D), lambda i: (0, 0)),
        ],
        out_specs=[
            pl.BlockSpec((_N, _E), lambda i: (0, 0)),
            pl.BlockSpec((1, 1), lambda i: (0, 0)),
        ],
        out_shape=[
            jax.ShapeDtypeStruct((_N, _E), jnp.float32),
            jax.ShapeDtypeStruct((1, 1), jnp.float32),
        ],
        scratch_shapes=[
            pltpu.SMEM((16,), jnp.int32),
            pltpu.VMEM((_N, _E), jnp.float32),
            pltpu.VMEM((_E, _N), jnp.float32),
            pltpu.VMEM((_E, _E), jnp.float32),
        ],
    )(hidden_states, W)
    return fw, jnp.reshape(lsim, ())
